# baseline (speedup 1.0000x reference)
import functools

import jax
import jax.numpy as jnp
from jax import lax
from jax.experimental import pallas as pl
from jax.experimental.pallas import tpu as pltpu
from jax.experimental.pallas import tpu_sc as plsc

B, T, C = 4, 8192, 64
D_MODEL = 1024
SPD = 86400
PERIODS = (SPD, SPD // 2, SPD // 3, SPD // 4, SPD // 6)
NT = len(PERIODS)
SUB = D_MODEL // NT
SUBP = 256
N_TOK = B * T

NC, NS = 2, 16
NW = NC * NS
TOK_W = N_TOK // NW
CHUNK = 128
NCHUNK = TOK_W // CHUNK
VPR = 128 // 16


def _sc_gather_build(period):
    mesh = plsc.VectorSubcoreMesh(core_axis_name="c", subcore_axis_name="s")

    @functools.partial(
        pl.kernel,
        out_type=jax.ShapeDtypeStruct((N_TOK, 128), jnp.int32),
        mesh=mesh,
        scratch_types=[
            pltpu.VMEM((NCHUNK, CHUNK), jnp.int32),
            pltpu.VMEM((NCHUNK, CHUNK), jnp.int32),
            pltpu.VMEM((3, CHUNK, 128), jnp.int32),
            pltpu.SemaphoreType.DMA((3,)),
        ],
    )
    def sc_gather(ti_hbm, tbl, out_hbm, raw_v, idx_v, rows_v, sem):
        wid = lax.axis_index("s") * NC + lax.axis_index("c")
        pltpu.sync_copy(ti_hbm.at[wid], raw_v)
        base = wid * TOK_W
        pvec = jnp.full((16,), period, dtype=jnp.int32)

        def mod_body(j, _):
            r = j // VPR
            col = (j % VPR) * 16
            idx_v[r, pl.ds(col, 16)] = lax.rem(raw_v[r, pl.ds(col, 16)], pvec)
            return 0

        lax.fori_loop(0, NCHUNK * VPR, mod_body, 0)

        def fire(c):
            p = c % 3
            pltpu.async_copy(tbl.at[idx_v.at[c]], rows_v.at[p], sem.at[p])

        fire(0)
        fire(1)

        def chunk_body(c, _):
            @pl.when(c + 2 < NCHUNK)
            def _():
                fire(c + 2)

            p = c % 3
            pltpu.make_async_copy(
                tbl.at[idx_v.at[c]], rows_v.at[p], sem.at[p]).wait()
            pltpu.sync_copy(
                rows_v.at[p], out_hbm.at[pl.ds(base + c * CHUNK, CHUNK), :])
            return 0

        lax.fori_loop(0, NCHUNK, chunk_body, 0)

    return sc_gather


_sc_gathers = [_sc_gather_build(p) for p in PERIODS]

TM = 2048


def _mm_body(a0, a1, a2, a3, a4, w_ref, b_ref, o_ref):
    acc = jnp.broadcast_to(b_ref[...], (TM, D_MODEL)).astype(jnp.float32)
    himask = jnp.int32(-65536)
    for i, a_ref in enumerate((a0, a1, a2, a3, a4)):
        word = a_ref[...]
        lo = lax.bitcast_convert_type(
            lax.shift_left(word, 16), jnp.float32).astype(jnp.bfloat16)
        hi = lax.bitcast_convert_type(
            lax.bitwise_and(word, himask), jnp.float32).astype(jnp.bfloat16)
        acc += jnp.dot(lo, w_ref[i, 0], preferred_element_type=jnp.float32)
        acc += jnp.dot(hi, w_ref[i, 1], preferred_element_type=jnp.float32)
    o_ref[...] = acc


def _tc_project(combs, wp4, bp2):
    return pl.pallas_call(
        _mm_body,
        grid=(N_TOK // TM,),
        in_specs=[pl.BlockSpec((TM, 128), lambda m: (m, 0))] * NT + [
            pl.BlockSpec((NT, 2, 128, D_MODEL), lambda m: (0, 0, 0, 0)),
            pl.BlockSpec((1, D_MODEL), lambda m: (0, 0)),
        ],
        out_specs=pl.BlockSpec((TM, D_MODEL), lambda m: (m, 0)),
        out_shape=jax.ShapeDtypeStruct((N_TOK, D_MODEL), jnp.float32),
        compiler_params=pltpu.CompilerParams(
            dimension_semantics=("arbitrary",),
        ),
    )(*combs, wp4, bp2)


def _xla_repack(t):
    lo = t[:, 0:128].astype(jnp.bfloat16)
    hi = jnp.pad(t[:, 128:SUB], ((0, 0), (0, SUBP - SUB))).astype(jnp.bfloat16)
    lo32 = lax.bitcast_convert_type(lo, jnp.uint16).astype(jnp.uint32)
    hi32 = lax.bitcast_convert_type(hi, jnp.uint16).astype(jnp.uint32)
    word = lax.bitwise_or(lo32, lax.shift_left(hi32, jnp.uint32(16)))
    return lax.bitcast_convert_type(word, jnp.int32)


def kernel(x, time_indices, table0, table1, table2, table3, table4, Wp, bp):
    del x
    ti = time_indices.reshape(-1).astype(jnp.int32).reshape(NW, NCHUNK, CHUNK)
    tabs = (table0, table1, table2, table3, table4)
    combs = []
    for i, t in enumerate(tabs):
        combs.append(_sc_gathers[i](ti, _xla_repack(t)))
    wp4 = jnp.pad(Wp.reshape(NT, SUB, D_MODEL),
                  ((0, 0), (0, SUBP - SUB), (0, 0)))
    wp4 = wp4.reshape(NT, 2, 128, D_MODEL).astype(jnp.bfloat16)
    out = _tc_project(combs, wp4, bp.reshape(1, D_MODEL))
    return out.reshape(B, T, D_MODEL)
